# baseline (speedup 1.0000x reference)
import functools

import jax
import jax.numpy as jnp
from jax import lax
from jax.experimental import pallas as pl
from jax.experimental.pallas import tpu as pltpu
from jax.experimental.pallas import tpu_sc as plsc

N = 10000
E = 320000
D_IN = 128
D_EDGE = 4
HID = 32
MSG = 128
OUT = 128
CAT = D_IN + MSG
DIM1 = 214
DIM2 = 172

W = 48
NC = 2
NS = 16
NW = NC * NS
EPW = E // NW
CH = 80
NCH = EPW // CH
NP = 10240
RPT = NP // NS
NB = 5

BE = 2560
BN = 2000

_HIGH = jax.lax.Precision.HIGHEST


def _leaky(v):
    return jnp.where(v > 0, v, 0.01 * v)


def _edge_body(ea_ref, w1_ref, b1_ref, w2_ref, b2_ref, out_ref):
    a = ea_ref[...]
    h = lax.dot_general(a, w1_ref[...], (((1,), (0,)), ((), ())),
                        preferred_element_type=jnp.float32) + b1_ref[...]
    h = _leaky(h)
    h = jnp.dot(h, w2_ref[...],
                preferred_element_type=jnp.float32) + b2_ref[...]
    h = _leaky(h)
    ones = jnp.ones((BE, 1), jnp.float32)
    pad = jnp.zeros((BE, W - HID - 1), jnp.float32)
    out_ref[...] = jnp.concatenate([h, ones, pad], axis=1)


def _edge_mlp(edge_attr, w1t, b1, w2t, b2):
    return pl.pallas_call(
        _edge_body,
        grid=(E // BE,),
        in_specs=[
            pl.BlockSpec((BE, D_EDGE), lambda i: (i, 0)),
            pl.BlockSpec((D_EDGE, HID), lambda i: (0, 0)),
            pl.BlockSpec((1, HID), lambda i: (0, 0)),
            pl.BlockSpec((HID, HID), lambda i: (0, 0)),
            pl.BlockSpec((1, HID), lambda i: (0, 0)),
        ],
        out_specs=pl.BlockSpec((BE, W), lambda i: (i, 0)),
        out_shape=jax.ShapeDtypeStruct((E, W), jnp.float32),
    )(edge_attr, w1t, b1, w2t, b2)


@functools.cache
def _segment_sum_sc_fn():
    mesh = plsc.VectorSubcoreMesh(core_axis_name="c", subcore_axis_name="s")

    @functools.partial(
        pl.kernel,
        mesh=mesh,
        out_type=jax.ShapeDtypeStruct((NC, NP, W), jnp.float32),
        compiler_params=pltpu.CompilerParams(use_tc_tiling_on_sc=False),
        scratch_types=[
            pltpu.VMEM((NCH, CH), jnp.int32),
            pltpu.VMEM((NB, CH, W), jnp.float32),
            pltpu.VMEM_SHARED((NP, W), jnp.float32),
            pltpu.SemaphoreType.DMA((NB,)),
            pltpu.SemaphoreType.DMA((NB,)),
        ],
    )
    def _segment_sum_sc(h2p_hbm, dst_hbm, zeros_hbm, out_hbm,
                        idx_v, buf_v, acc_sh, gsem, ssem):
        cid = lax.axis_index("c")
        sid = lax.axis_index("s")
        wid = sid * NC + cid
        pltpu.sync_copy(zeros_hbm, buf_v.at[0])

        def zinit(r, carry):
            pltpu.sync_copy(buf_v.at[0],
                            acc_sh.at[pl.ds(sid * RPT + r * CH, CH)])
            return carry

        lax.fori_loop(0, RPT // CH, zinit, 0)
        plsc.subcore_barrier()
        pltpu.sync_copy(dst_hbm.at[wid], idx_v)
        base = wid * EPW

        def gather(c, k):
            pltpu.make_async_copy(
                h2p_hbm.at[pl.ds(base + c * CH, CH)], buf_v.at[k],
                gsem.at[k]).start()

        def gather_wait(k):
            pltpu.make_async_copy(
                h2p_hbm.at[pl.ds(0, CH)], buf_v.at[k], gsem.at[k]).wait()

        def scatter(c, k):
            pltpu.make_async_copy(
                buf_v.at[k], acc_sh.at[idx_v.at[c]], ssem.at[k]
            ).start(add=True)

        def scatter_wait(k):
            pltpu.make_async_copy(
                buf_v.at[k], acc_sh.at[pl.ds(0, CH)], ssem.at[k]).wait()

        for k in range(NB):
            gather(k, k)

        def rounds(i, carry):
            for k in range(NB):
                gather_wait(k)
                scatter(i * NB + k, k)
            for k in range(NB):
                c = i * NB + k

                @pl.when(c + NB < NCH)
                def _():
                    scatter_wait(k)
                    gather(c + NB, k)
            return carry

        lax.fori_loop(0, NCH // NB, rounds, 0)
        for k in range(NB):
            scatter_wait(k)
        plsc.subcore_barrier()

        def publish(r, carry):
            pltpu.sync_copy(acc_sh.at[pl.ds(sid * RPT + r * CH, CH)],
                            buf_v.at[0])
            pltpu.sync_copy(buf_v.at[0],
                            out_hbm.at[cid, pl.ds(sid * RPT + r * CH, CH)])
            return carry

        lax.fori_loop(0, RPT // CH, publish, 0)

    return _segment_sum_sc


def _update_body(x_ref, p_ref, wext_ref, g_ref, bt_ref,
                 w1_ref, b1_ref, w2_ref, b2_ref, w3_ref, b3_ref, out_ref):
    p = p_ref[0] + p_ref[1]
    aggr = jnp.dot(p, wext_ref[...], precision=_HIGH,
                   preferred_element_type=jnp.float32)
    cat = jnp.concatenate([x_ref[...], aggr], axis=1)
    mu = jnp.mean(cat, axis=1, keepdims=True)
    var = jnp.mean((cat - mu) ** 2, axis=1, keepdims=True)
    normed = (cat - mu) * lax.rsqrt(var + 1e-5) * g_ref[...] + bt_ref[...]
    h = jnp.dot(normed, w1_ref[...], precision=_HIGH,
                preferred_element_type=jnp.float32) + b1_ref[...]
    h = _leaky(h)
    h = jnp.dot(h, w2_ref[...], precision=_HIGH,
                preferred_element_type=jnp.float32) + b2_ref[...]
    h = _leaky(h)
    out_ref[...] = jnp.dot(h, w3_ref[...], precision=_HIGH,
                           preferred_element_type=jnp.float32) + b3_ref[...]


def _update(x, partials, wext, gamma, beta, w1t, b1, w2t, b2, w3t, b3):
    full = lambda r, c: pl.BlockSpec((r, c), lambda i: (0, 0))
    return pl.pallas_call(
        _update_body,
        grid=(N // BN,),
        in_specs=[
            pl.BlockSpec((BN, D_IN), lambda i: (i, 0)),
            pl.BlockSpec((NC, BN, W), lambda i: (0, i, 0)),
            full(W, OUT),
            full(1, CAT),
            full(1, CAT),
            full(CAT, DIM1),
            full(1, DIM1),
            full(DIM1, DIM2),
            full(1, DIM2),
            full(DIM2, OUT),
            full(1, OUT),
        ],
        out_specs=pl.BlockSpec((BN, OUT), lambda i: (i, 0)),
        out_shape=jax.ShapeDtypeStruct((N, OUT), jnp.float32),
    )(x, partials, wext, gamma, beta, w1t, b1, w2t, b2, w3t, b3)


def kernel(x, edge_index, edge_attr, mW1, mb1, mW2, mb2, mW3, mb3,
           gamma, beta, uW1, ub1, uW2, ub2, uW3, ub3):
    h2p = _edge_mlp(edge_attr, mW1.T, mb1[None, :], mW2.T, mb2[None, :])
    dst3 = edge_index[1].astype(jnp.int32).reshape(NW, NCH, CH)
    zeros = jnp.zeros((CH, W), jnp.float32)
    partials = _segment_sum_sc_fn()(h2p, dst3, zeros)[:, :N]
    wext = jnp.concatenate(
        [mW3.T, mb3[None, :], jnp.zeros((W - HID - 1, MSG), jnp.float32)],
        axis=0)
    return _update(x, partials, wext, gamma[None, :], beta[None, :],
                   uW1.T, ub1[None, :], uW2.T, ub2[None, :],
                   uW3.T, ub3[None, :])


# speedup vs baseline: 1.4225x; 1.4225x over previous
import functools

import jax
import jax.numpy as jnp
from jax import lax
from jax.experimental import pallas as pl
from jax.experimental.pallas import tpu as pltpu
from jax.experimental.pallas import tpu_sc as plsc

N = 10000
E = 320000
D_IN = 128
D_EDGE = 4
HID = 32
MSG = 128
OUT = 128
CAT = D_IN + MSG
DIM1 = 214
DIM2 = 172

W = 48
NC = 2
NS = 16
NW = NC * NS
EPW = E // NW
CH = 80
NCH = EPW // CH
NP = 10240
RPT = NP // NS
NB = 5

BE = 2560
BN = 2000

def _leaky(v):
    return jnp.where(v > 0, v, 0.01 * v)


def _edge_body(ea_ref, w1_ref, b1_ref, w2_ref, b2_ref, out_ref):
    a = ea_ref[...]
    h = lax.dot_general(a, w1_ref[...], (((0,), (0,)), ((), ())),
                        preferred_element_type=jnp.float32) + b1_ref[...]
    h = _leaky(h)
    h = jnp.dot(h, w2_ref[...],
                preferred_element_type=jnp.float32) + b2_ref[...]
    h = _leaky(h)
    ones = jnp.ones((BE, 1), jnp.float32)
    pad = jnp.zeros((BE, W - HID - 1), jnp.float32)
    out_ref[...] = jnp.concatenate([h, ones, pad], axis=1)


def _edge_mlp(edge_attr_t, w1t, b1, w2t, b2):
    return pl.pallas_call(
        _edge_body,
        grid=(E // BE,),
        in_specs=[
            pl.BlockSpec((D_EDGE, BE), lambda i: (0, i)),
            pl.BlockSpec((D_EDGE, HID), lambda i: (0, 0)),
            pl.BlockSpec((1, HID), lambda i: (0, 0)),
            pl.BlockSpec((HID, HID), lambda i: (0, 0)),
            pl.BlockSpec((1, HID), lambda i: (0, 0)),
        ],
        out_specs=pl.BlockSpec((BE, W), lambda i: (i, 0)),
        out_shape=jax.ShapeDtypeStruct((E, W), jnp.float32),
    )(edge_attr_t, w1t, b1, w2t, b2)


@functools.cache
def _segment_sum_sc_fn():
    mesh = plsc.VectorSubcoreMesh(core_axis_name="c", subcore_axis_name="s")

    @functools.partial(
        pl.kernel,
        mesh=mesh,
        out_type=jax.ShapeDtypeStruct((NC, NP, W), jnp.float32),
        compiler_params=pltpu.CompilerParams(use_tc_tiling_on_sc=False),
        scratch_types=[
            pltpu.VMEM((NCH, CH), jnp.int32),
            pltpu.VMEM((NB, CH, W), jnp.float32),
            pltpu.VMEM_SHARED((NP, W), jnp.float32),
            pltpu.SemaphoreType.DMA((NB,)),
            pltpu.SemaphoreType.DMA((NB,)),
        ],
    )
    def _segment_sum_sc(h2p_hbm, dst_hbm, zeros_hbm, out_hbm,
                        idx_v, buf_v, acc_sh, gsem, ssem):
        cid = lax.axis_index("c")
        sid = lax.axis_index("s")
        wid = sid * NC + cid
        pltpu.sync_copy(zeros_hbm, buf_v.at[0])

        def zinit(r, carry):
            pltpu.sync_copy(buf_v.at[0],
                            acc_sh.at[pl.ds(sid * RPT + r * CH, CH)])
            return carry

        lax.fori_loop(0, RPT // CH, zinit, 0)
        plsc.subcore_barrier()
        pltpu.sync_copy(dst_hbm.at[wid], idx_v)
        base = wid * EPW

        def gather(c, k):
            pltpu.make_async_copy(
                h2p_hbm.at[pl.ds(base + c * CH, CH)], buf_v.at[k],
                gsem.at[k]).start()

        def gather_wait(k):
            pltpu.make_async_copy(
                h2p_hbm.at[pl.ds(0, CH)], buf_v.at[k], gsem.at[k]).wait()

        def scatter(c, k):
            pltpu.make_async_copy(
                buf_v.at[k], acc_sh.at[idx_v.at[c]], ssem.at[k]
            ).start(add=True)

        def scatter_wait(k):
            pltpu.make_async_copy(
                buf_v.at[k], acc_sh.at[pl.ds(0, CH)], ssem.at[k]).wait()

        for k in range(NB):
            gather(k, k)

        def rounds(i, carry):
            for k in range(NB):
                gather_wait(k)
                scatter(i * NB + k, k)
            for k in range(NB):
                c = i * NB + k

                @pl.when(c + NB < NCH)
                def _():
                    scatter_wait(k)
                    gather(c + NB, k)
            return carry

        lax.fori_loop(0, NCH // NB, rounds, 0)
        for k in range(NB):
            scatter_wait(k)
        plsc.subcore_barrier()

        def publish(r, carry):
            pltpu.sync_copy(acc_sh.at[pl.ds(sid * RPT + r * CH, CH)],
                            buf_v.at[0])
            pltpu.sync_copy(buf_v.at[0],
                            out_hbm.at[cid, pl.ds(sid * RPT + r * CH, CH)])
            return carry

        lax.fori_loop(0, RPT // CH, publish, 0)

    return _segment_sum_sc


def _update_body(x_ref, p_ref, wext_ref, g_ref, bt_ref,
                 w1_ref, b1_ref, w2_ref, b2_ref, w3_ref, b3_ref, out_ref):
    p = p_ref[0] + p_ref[1]
    aggr = jnp.dot(p, wext_ref[...],
                   preferred_element_type=jnp.float32)
    cat = jnp.concatenate([x_ref[...], aggr], axis=1)
    mu = jnp.mean(cat, axis=1, keepdims=True)
    var = jnp.mean((cat - mu) ** 2, axis=1, keepdims=True)
    normed = (cat - mu) * lax.rsqrt(var + 1e-5) * g_ref[...] + bt_ref[...]
    h = jnp.dot(normed, w1_ref[...],
                preferred_element_type=jnp.float32) + b1_ref[...]
    h = _leaky(h)
    h = jnp.dot(h, w2_ref[...],
                preferred_element_type=jnp.float32) + b2_ref[...]
    h = _leaky(h)
    out_ref[...] = jnp.dot(h, w3_ref[...],
                           preferred_element_type=jnp.float32) + b3_ref[...]


def _update(x, partials, wext, gamma, beta, w1t, b1, w2t, b2, w3t, b3):
    full = lambda r, c: pl.BlockSpec((r, c), lambda i: (0, 0))
    return pl.pallas_call(
        _update_body,
        grid=(N // BN,),
        in_specs=[
            pl.BlockSpec((BN, D_IN), lambda i: (i, 0)),
            pl.BlockSpec((NC, BN, W), lambda i: (0, i, 0)),
            full(W, OUT),
            full(1, CAT),
            full(1, CAT),
            full(CAT, DIM1),
            full(1, DIM1),
            full(DIM1, DIM2),
            full(1, DIM2),
            full(DIM2, OUT),
            full(1, OUT),
        ],
        out_specs=pl.BlockSpec((BN, OUT), lambda i: (i, 0)),
        out_shape=jax.ShapeDtypeStruct((N, OUT), jnp.float32),
    )(x, partials, wext, gamma, beta, w1t, b1, w2t, b2, w3t, b3)


def kernel(x, edge_index, edge_attr, mW1, mb1, mW2, mb2, mW3, mb3,
           gamma, beta, uW1, ub1, uW2, ub2, uW3, ub3):
    h2p = _edge_mlp(edge_attr.T, mW1.T, mb1[None, :], mW2.T, mb2[None, :])
    dst3 = edge_index[1].astype(jnp.int32).reshape(NW, NCH, CH)
    zeros = jnp.zeros((CH, W), jnp.float32)
    partials = _segment_sum_sc_fn()(h2p, dst3, zeros)[:, :N]
    wext = jnp.concatenate(
        [mW3.T, mb3[None, :], jnp.zeros((W - HID - 1, MSG), jnp.float32)],
        axis=0)
    return _update(x, partials, wext, gamma[None, :], beta[None, :],
                   uW1.T, ub1[None, :], uW2.T, ub2[None, :],
                   uW3.T, ub3[None, :])
